# baseline (speedup 1.0000x reference)
import functools
import jax
import jax.numpy as jnp
from jax import lax
from jax.experimental import pallas as pl
from jax.experimental.pallas import tpu as pltpu
from jax.experimental.pallas import tpu_sc as plsc

N_NODES = 2249
N_DRUG = 2033
NHID = 128
ALPHA = 0.9
N_EDGES = 143936
BATCH = 16384

NP = 2304
BLK = 256
E_T = 9088
EP = 16 * E_T
NROWS_I = 71
R = 576
BUF_MAIN = R * NP
BUFLEN = BUF_MAIN + 128
STRIPE = BUF_MAIN // 16
ZV = STRIPE // 4
DST_PAD = 4000

_SC_MESH = plsc.VectorSubcoreMesh(core_axis_name="c", subcore_axis_name="s")


def _sc_build_body(srcs, dsts, ones_h, zeros_h, c_out,
                   src_v, dst_v, idx1d, ones_v, buf):
    core = lax.axis_index("c")
    sub = lax.axis_index("s")
    pltpu.sync_copy(ones_h, ones_v)
    eoff = sub * E_T
    for g in range(3):
        pltpu.sync_copy(srcs.at[pl.ds(g * EP + eoff, E_T)], src_v)
        pltpu.sync_copy(dsts.at[pl.ds(g * EP + eoff, E_T)], dst_v)
        for ch_i in range(2):
            chunk = core * 2 + ch_i
            base = chunk * R
            pltpu.sync_copy(zeros_h, buf.at[pl.ds(sub * STRIPE, STRIPE)])
            plsc.subcore_barrier()

            def idx_body(b, carry):
                for k in range(8):
                    e = b * 128 + k * 16
                    s = src_v[pl.ds(e, 16)]
                    d = dst_v[pl.ds(e, 16)]
                    rel = d - base
                    ok = (rel >= 0) & (rel < R)
                    flat = rel * NP + s
                    flat = jnp.where(ok, flat, BUF_MAIN)
                    idx1d[pl.ds(e, 16)] = flat
                return carry
            lax.fori_loop(0, NROWS_I, idx_body, 0)

            pltpu.sync_copy(ones_v, buf.at[idx1d], add=True)
            plsc.subcore_barrier()

            dst_off = g * NP * NP + base * NP + sub * STRIPE
            pltpu.sync_copy(buf.at[pl.ds(sub * STRIPE, STRIPE)],
                            c_out.at[pl.ds(dst_off, STRIPE)])
            plsc.subcore_barrier()


def _sc_build(srcs, dsts):
    ones_h = jnp.ones((E_T,), jnp.float32)
    zeros_h = jnp.zeros((STRIPE,), jnp.float32)
    k = pl.kernel(
        _sc_build_body,
        out_type=jax.ShapeDtypeStruct((3 * NP * NP,), jnp.float32),
        mesh=_SC_MESH,
        scratch_types=[
            pltpu.VMEM((E_T,), jnp.int32),
            pltpu.VMEM((E_T,), jnp.int32),
            pltpu.VMEM((E_T,), jnp.int32),
            pltpu.VMEM((E_T,), jnp.float32),
            pltpu.VMEM_SHARED((BUFLEN,), jnp.float32),
        ],
    )
    return k(srcs, dsts, ones_h, zeros_h).reshape(3, NP, NP)


def _sc_gather_body(nf, s0r, s1r, dr, di, idx_s, rows_v, sem):
    core = lax.axis_index("c")
    sub = lax.axis_index("s")
    wid = sub * 2 + core
    pltpu.sync_copy(s0r.at[pl.ds(wid * 4, 4)], idx_s.at[pl.ds(0, 4)])
    pltpu.sync_copy(s1r.at[pl.ds(wid * 4, 4)], idx_s.at[pl.ds(4, 4)])

    def addb(i, carry):
        r = 4 + i // 8
        c = (i % 8) * 16
        idx_s[r, pl.ds(c, 16)] = idx_s[r, pl.ds(c, 16)] + N_DRUG
        return carry
    lax.fori_loop(0, 32, addb, 0)

    for t, out in ((0, dr), (1, di)):
        for b in range(4):
            pltpu.async_copy(nf.at[idx_s.at[t * 4 + b]], rows_v, sem).wait()
            pltpu.sync_copy(rows_v, out.at[pl.ds(wid * 512 + b * 128, 128)])


def _sc_gather(nf, s0r, s1r):
    k = pl.kernel(
        _sc_gather_body,
        out_type=(jax.ShapeDtypeStruct((BATCH, NHID), jnp.float32),
                  jax.ShapeDtypeStruct((BATCH, NHID), jnp.float32)),
        mesh=_SC_MESH,
        scratch_types=[
            pltpu.VMEM((8, 128), jnp.int32),
            pltpu.VMEM((128, NHID), jnp.float32),
            pltpu.SemaphoreType.DMA,
        ],
    )
    return k(nf, s0r, s1r)


def _h1_body(a_ref, w_ref, b_ref, o_ref):
    w = w_ref[0] + w_ref[1]
    b = b_ref[0] + b_ref[1]
    o_ref[...] = jnp.dot(a_ref[...], w, preferred_element_type=jnp.float32, precision=lax.Precision.HIGHEST) + b


def _mm_h1(adj_p, w_p, b):
    return pl.pallas_call(
        _h1_body,
        grid=(NP // BLK,),
        in_specs=[
            pl.BlockSpec((BLK, NP), lambda i: (i, 0)),
            pl.BlockSpec((2, NP, NHID), lambda i: (0, 0, 0)),
            pl.BlockSpec((2, 1, NHID), lambda i: (0, 0, 0)),
        ],
        out_specs=pl.BlockSpec((BLK, NHID), lambda i: (i, 0)),
        out_shape=jax.ShapeDtypeStruct((NP, NHID), jnp.float32),
    )(adj_p, w_p, b)


def _agg_fused_body(c_ref, m_ref, w_ref, b_ref, x_ref, mn_ref):
    cb = c_ref[...]
    deg = jnp.sum(cb, axis=1, keepdims=True)
    invd = 0.5 / jnp.maximum(deg, 1.0)
    y = jnp.dot(cb, m_ref[...], preferred_element_type=jnp.float32, precision=lax.Precision.HIGHEST) * invd
    x = jnp.maximum(y, 0.0)
    x_ref[...] = x
    w = w_ref[0] + w_ref[1]
    mn_ref[...] = jnp.dot(x, w, preferred_element_type=jnp.float32, precision=lax.Precision.HIGHEST) + (b_ref[0] + b_ref[1])


def _agg_fused(c, m, w, b):
    return pl.pallas_call(
        _agg_fused_body,
        grid=(NP // BLK,),
        in_specs=[
            pl.BlockSpec((BLK, NP), lambda i: (i, 0)),
            pl.BlockSpec((NP, NHID), lambda i: (0, 0)),
            pl.BlockSpec((2, NHID, NHID), lambda i: (0, 0, 0)),
            pl.BlockSpec((2, 1, NHID), lambda i: (0, 0, 0)),
        ],
        out_specs=[
            pl.BlockSpec((BLK, NHID), lambda i: (i, 0)),
            pl.BlockSpec((BLK, NHID), lambda i: (i, 0)),
        ],
        out_shape=[
            jax.ShapeDtypeStruct((NP, NHID), jnp.float32),
            jax.ShapeDtypeStruct((NP, NHID), jnp.float32),
        ],
    )(c, m, w, b)


def _agg_plain_body(c_ref, m_ref, x_ref):
    cb = c_ref[...]
    deg = jnp.sum(cb, axis=1, keepdims=True)
    invd = 0.5 / jnp.maximum(deg, 1.0)
    y = jnp.dot(cb, m_ref[...], preferred_element_type=jnp.float32, precision=lax.Precision.HIGHEST) * invd
    x_ref[...] = jnp.maximum(y, 0.0)


def _agg_plain(c, m):
    return pl.pallas_call(
        _agg_plain_body,
        grid=(NP // BLK,),
        in_specs=[
            pl.BlockSpec((BLK, NP), lambda i: (i, 0)),
            pl.BlockSpec((NP, NHID), lambda i: (0, 0)),
        ],
        out_specs=pl.BlockSpec((BLK, NHID), lambda i: (i, 0)),
        out_shape=jax.ShapeDtypeStruct((NP, NHID), jnp.float32),
    )(c, m)


def _finalize_body(gw, c_ref, x1_ref, x2_ref, x3_ref, pw1_ref, pb1_ref,
                   pw2_ref, pb2_ref, o_ref):
    i = pl.program_id(0)
    cb = c_ref[...]
    deg = jnp.sum(cb, axis=1, keepdims=True)
    invd = 1.0 / jnp.maximum(deg, 1.0)
    pw1 = pw1_ref[...]
    pw2 = pw2_ref[...]
    pb1 = pb1_ref[0, 0]
    pb2 = pb2_ref[0, 0]
    xs_blk = []
    pcols = []
    gcols = []
    for x_ref in (x1_ref, x2_ref, x3_ref):
        xf = x_ref[...]
        xb = x_ref[pl.ds(i * BLK, BLK), :]
        xs_blk.append(xb)
        pcols.append(jnp.sum(xf * pw2, axis=1, keepdims=True) + pb2)
        gcols.append(jnp.sum(xb * pw1, axis=1, keepdims=True) + pb1)
    p_full = jnp.concatenate(pcols, axis=1)
    sl = jnp.dot(cb, p_full, preferred_element_type=jnp.float32, precision=lax.Precision.HIGHEST) * invd
    g = jnp.concatenate(gcols, axis=1)
    w = ALPHA * g + (1.0 - ALPHA) * sl
    m = jnp.max(w, axis=1, keepdims=True)
    e = jnp.exp(w - m)
    wn = e / jnp.sum(e, axis=1, keepdims=True)
    o_ref[...] = gw * (wn[:, 0:1] * xs_blk[0]
                       + wn[:, 1:2] * xs_blk[1]
                       + wn[:, 2:3] * xs_blk[2])


def _finalize(gw, c, x1, x2, x3, pw1, pb1, pw2, pb2):
    return pl.pallas_call(
        functools.partial(_finalize_body, gw),
        grid=(NP // BLK,),
        in_specs=[
            pl.BlockSpec((BLK, NP), lambda i: (i, 0)),
            pl.BlockSpec((NP, NHID), lambda i: (0, 0)),
            pl.BlockSpec((NP, NHID), lambda i: (0, 0)),
            pl.BlockSpec((NP, NHID), lambda i: (0, 0)),
            pl.BlockSpec((1, NHID), lambda i: (0, 0)),
            pl.BlockSpec((1, 1), lambda i: (0, 0)),
            pl.BlockSpec((1, NHID), lambda i: (0, 0)),
            pl.BlockSpec((1, 1), lambda i: (0, 0)),
        ],
        out_specs=pl.BlockSpec((BLK, NHID), lambda i: (i, 0)),
        out_shape=jax.ShapeDtypeStruct((NP, NHID), jnp.float32),
    )(c, x1, x2, x3, pw1, pb1, pw2, pb2)


def _combine_body(a_ref, b_ref, c_ref, o_ref):
    o_ref[...] = a_ref[...] + b_ref[...] + c_ref[...]


def _combine(a, b, c):
    return pl.pallas_call(
        _combine_body,
        grid=(NP // BLK,),
        in_specs=[pl.BlockSpec((BLK, NHID), lambda i: (i, 0))] * 3,
        out_specs=pl.BlockSpec((BLK, NHID), lambda i: (i, 0)),
        out_shape=jax.ShapeDtypeStruct((NP, NHID), jnp.float32),
    )(a, b, c)


BMLP = 512


def _mlp_body(dr_ref, di_ref, w0_ref, b0_ref, w1_ref, b1_ref, w2_ref, b2_ref,
              w3_ref, b3_ref, emb_ref, o_ref):
    e = dr_ref[...] * di_ref[...]
    emb_ref[...] = e
    h = jnp.maximum(jnp.dot(e, w0_ref[...], preferred_element_type=jnp.float32, precision=lax.Precision.HIGHEST)
                    + b0_ref[...], 0.0)
    h = jnp.maximum(jnp.dot(h, w1_ref[...], preferred_element_type=jnp.float32, precision=lax.Precision.HIGHEST)
                    + b1_ref[...], 0.0)
    h = jnp.maximum(jnp.dot(h, w2_ref[...], preferred_element_type=jnp.float32, precision=lax.Precision.HIGHEST)
                    + b2_ref[...], 0.0)
    o_ref[...] = jnp.dot(h, w3_ref[...], preferred_element_type=jnp.float32, precision=lax.Precision.HIGHEST) + b3_ref[...]


def _mlp(dr, di, w0, b0, w1, b1, w2, b2, w3p, b3p):
    full = lambda r, c: pl.BlockSpec((r, c), lambda i: (0, 0))
    return pl.pallas_call(
        _mlp_body,
        grid=(BATCH // BMLP,),
        in_specs=[
            pl.BlockSpec((BMLP, NHID), lambda i: (i, 0)),
            pl.BlockSpec((BMLP, NHID), lambda i: (i, 0)),
            full(NHID, 1024), full(1, 1024),
            full(1024, 512), full(1, 512),
            full(512, 256), full(1, 256),
            full(256, 128), full(1, 128),
        ],
        out_specs=[
            pl.BlockSpec((BMLP, NHID), lambda i: (i, 0)),
            pl.BlockSpec((BMLP, 128), lambda i: (i, 0)),
        ],
        out_shape=[
            jax.ShapeDtypeStruct((BATCH, NHID), jnp.float32),
            jax.ShapeDtypeStruct((BATCH, 128), jnp.float32),
        ],
    )(dr, di, w0, b0, w1, b1, w2, b2, w3p, b3p)


def kernel(sample, adj, adj_edge_index1, adj_edge_index2, adj_edge_index3,
           W_gc1, b_gc1, W_gc2, b_gc2, W_gc3, b_gc3,
           pool_w1, pool_b1, pool_w2, pool_b2,
           mlp_w0, mlp_b0, mlp_w1, mlp_b1, mlp_w2, mlp_b2, mlp_w3, mlp_b3):
    f32 = jnp.float32

    srcs = jnp.concatenate(
        [jnp.pad(ei[0].astype(jnp.int32), (0, EP - N_EDGES))
         for ei in (adj_edge_index1, adj_edge_index2, adj_edge_index3)])
    dsts = jnp.concatenate(
        [jnp.pad(ei[1].astype(jnp.int32), (0, EP - N_EDGES),
                 constant_values=DST_PAD)
         for ei in (adj_edge_index1, adj_edge_index2, adj_edge_index3)])
    adj_p = jnp.pad(adj.astype(f32), ((0, NP - N_NODES), (0, NP - N_NODES)))
    w1_p = jnp.pad(W_gc1.astype(f32), ((0, 0), (0, NP - N_NODES), (0, 0)))
    b1r = b_gc1.reshape(2, 1, NHID).astype(f32)
    b2r = b_gc2.reshape(2, 1, NHID).astype(f32)
    b3r = b_gc3.reshape(2, 1, NHID).astype(f32)
    pw1 = pool_w1.reshape(1, NHID).astype(f32)
    pw2 = pool_w2.reshape(1, NHID).astype(f32)
    pb1 = pool_b1.reshape(1, 1).astype(f32)
    pb2 = pool_b2.reshape(1, 1).astype(f32)
    s0r = sample[:, 0].astype(jnp.int32).reshape(128, 128)
    s1r = sample[:, 1].astype(jnp.int32).reshape(128, 128)
    w3p = jnp.pad(mlp_w3.astype(f32), ((0, 0), (0, 126)))
    b3p = jnp.pad(mlp_b3.astype(f32), (0, 126)).reshape(1, 128)
    b0r = mlp_b0.reshape(1, 1024).astype(f32)
    b1m = mlp_b1.reshape(1, 512).astype(f32)
    b2m = mlp_b2.reshape(1, 256).astype(f32)

    c_all = _sc_build(srcs, dsts)

    h1 = _mm_h1(adj_p, w1_p, b1r)

    nfs = []
    for g, gw in enumerate((0.3, 0.35, 0.35)):
        cg = c_all[g]
        x1, m2 = _agg_fused(cg, h1, W_gc2.astype(f32), b2r)
        x2, m3 = _agg_fused(cg, m2, W_gc3.astype(f32), b3r)
        x3 = _agg_plain(cg, m3)
        nfs.append(_finalize(gw, cg, x1, x2, x3, pw1, pb1, pw2, pb2))
    nf = _combine(*nfs)

    dr_rows, di_rows = _sc_gather(nf, s0r, s1r)

    emb, outp = _mlp(dr_rows, di_rows, mlp_w0.astype(f32), b0r,
                     mlp_w1.astype(f32), b1m, mlp_w2.astype(f32), b2m, w3p, b3p)
    return emb, outp[:, :2]


# speedup vs baseline: 1.8269x; 1.8269x over previous
import functools
import jax
import jax.numpy as jnp
from jax import lax
from jax.experimental import pallas as pl
from jax.experimental.pallas import tpu as pltpu
from jax.experimental.pallas import tpu_sc as plsc

N_NODES = 2249
N_DRUG = 2033
NHID = 128
ALPHA = 0.9
N_EDGES = 143936
BATCH = 16384

NP = 2304
BLK = 256
E_T = 9088
EP = 16 * E_T
NROWS_I = 71
R = 576
BUF_MAIN = R * NP
BUFLEN = BUF_MAIN + 2048
STRIPE = BUF_MAIN // 16
ZV = STRIPE // 4
DST_PAD = 4000

_SC_MESH = plsc.VectorSubcoreMesh(core_axis_name="c", subcore_axis_name="s")


def _sc_build_body(srcs, dsts, ones_h, zeros_h, c_out,
                   src_v, dst_v, idx1d, ones_v, buf):
    core = lax.axis_index("c")
    sub = lax.axis_index("s")
    pltpu.sync_copy(ones_h, ones_v)
    eoff = sub * E_T
    lane = lax.iota(jnp.int32, 16)
    gbase = BUF_MAIN + sub * 128
    for g in range(3):
        pltpu.sync_copy(srcs.at[pl.ds(g * EP + eoff, E_T)], src_v)
        pltpu.sync_copy(dsts.at[pl.ds(g * EP + eoff, E_T)], dst_v)
        for ch_i in range(2):
            chunk = core * 2 + ch_i
            base = chunk * R
            pltpu.sync_copy(zeros_h, buf.at[pl.ds(sub * STRIPE, STRIPE)])
            plsc.subcore_barrier()

            def idx_body(b, carry):
                for k in range(8):
                    e = b * 128 + k * 16
                    s = src_v[pl.ds(e, 16)]
                    d = dst_v[pl.ds(e, 16)]
                    rel = d - base
                    ok = (rel >= 0) & (rel < R)
                    flat = rel * NP + s
                    flat = jnp.where(ok, flat, gbase + k * 16 + lane)
                    idx1d[pl.ds(e, 16)] = flat
                return carry
            lax.fori_loop(0, NROWS_I, idx_body, 0)

            pltpu.sync_copy(ones_v, buf.at[idx1d], add=True)
            plsc.subcore_barrier()

            dst_off = g * NP * NP + base * NP + sub * STRIPE
            pltpu.sync_copy(buf.at[pl.ds(sub * STRIPE, STRIPE)],
                            c_out.at[pl.ds(dst_off, STRIPE)])
            plsc.subcore_barrier()


def _sc_build(srcs, dsts):
    ones_h = jnp.ones((E_T,), jnp.float32)
    zeros_h = jnp.zeros((STRIPE,), jnp.float32)
    k = pl.kernel(
        _sc_build_body,
        out_type=jax.ShapeDtypeStruct((3 * NP * NP,), jnp.float32),
        mesh=_SC_MESH,
        scratch_types=[
            pltpu.VMEM((E_T,), jnp.int32),
            pltpu.VMEM((E_T,), jnp.int32),
            pltpu.VMEM((E_T,), jnp.int32),
            pltpu.VMEM((E_T,), jnp.float32),
            pltpu.VMEM_SHARED((BUFLEN,), jnp.float32),
        ],
    )
    return k(srcs, dsts, ones_h, zeros_h).reshape(3, NP, NP)


def _sc_gather_body(nf, s0r, s1r, dr, di, idx_s, rows_v, sem):
    core = lax.axis_index("c")
    sub = lax.axis_index("s")
    wid = sub * 2 + core
    pltpu.sync_copy(s0r.at[pl.ds(wid * 4, 4)], idx_s.at[pl.ds(0, 4)])
    pltpu.sync_copy(s1r.at[pl.ds(wid * 4, 4)], idx_s.at[pl.ds(4, 4)])

    def addb(i, carry):
        r = 4 + i // 8
        c = (i % 8) * 16
        idx_s[r, pl.ds(c, 16)] = idx_s[r, pl.ds(c, 16)] + N_DRUG
        return carry
    lax.fori_loop(0, 32, addb, 0)

    for t, out in ((0, dr), (1, di)):
        for b in range(4):
            pltpu.async_copy(nf.at[idx_s.at[t * 4 + b]], rows_v, sem).wait()
            pltpu.sync_copy(rows_v, out.at[pl.ds(wid * 512 + b * 128, 128)])


def _sc_gather(nf, s0r, s1r):
    k = pl.kernel(
        _sc_gather_body,
        out_type=(jax.ShapeDtypeStruct((BATCH, NHID), jnp.float32),
                  jax.ShapeDtypeStruct((BATCH, NHID), jnp.float32)),
        mesh=_SC_MESH,
        scratch_types=[
            pltpu.VMEM((8, 128), jnp.int32),
            pltpu.VMEM((128, NHID), jnp.float32),
            pltpu.SemaphoreType.DMA,
        ],
    )
    return k(nf, s0r, s1r)


def _h1_body(a_ref, w_ref, b_ref, o_ref):
    w = w_ref[0] + w_ref[1]
    b = b_ref[0] + b_ref[1]
    o_ref[...] = jnp.dot(a_ref[...], w, preferred_element_type=jnp.float32, precision=lax.Precision.HIGHEST) + b


def _mm_h1(adj_p, w_p, b):
    return pl.pallas_call(
        _h1_body,
        grid=(NP // BLK,),
        in_specs=[
            pl.BlockSpec((BLK, NP), lambda i: (i, 0)),
            pl.BlockSpec((2, NP, NHID), lambda i: (0, 0, 0)),
            pl.BlockSpec((2, 1, NHID), lambda i: (0, 0, 0)),
        ],
        out_specs=pl.BlockSpec((BLK, NHID), lambda i: (i, 0)),
        out_shape=jax.ShapeDtypeStruct((NP, NHID), jnp.float32),
    )(adj_p, w_p, b)


def _agg_fused_body(c_ref, m_ref, w_ref, b_ref, x_ref, mn_ref):
    cb = c_ref[...]
    deg = jnp.sum(cb, axis=1, keepdims=True)
    invd = 0.5 / jnp.maximum(deg, 1.0)
    y = jnp.dot(cb, m_ref[...], preferred_element_type=jnp.float32, precision=lax.Precision.HIGHEST) * invd
    x = jnp.maximum(y, 0.0)
    x_ref[...] = x
    w = w_ref[0] + w_ref[1]
    mn_ref[...] = jnp.dot(x, w, preferred_element_type=jnp.float32, precision=lax.Precision.HIGHEST) + (b_ref[0] + b_ref[1])


def _agg_fused(c, m, w, b):
    return pl.pallas_call(
        _agg_fused_body,
        grid=(NP // BLK,),
        in_specs=[
            pl.BlockSpec((BLK, NP), lambda i: (i, 0)),
            pl.BlockSpec((NP, NHID), lambda i: (0, 0)),
            pl.BlockSpec((2, NHID, NHID), lambda i: (0, 0, 0)),
            pl.BlockSpec((2, 1, NHID), lambda i: (0, 0, 0)),
        ],
        out_specs=[
            pl.BlockSpec((BLK, NHID), lambda i: (i, 0)),
            pl.BlockSpec((BLK, NHID), lambda i: (i, 0)),
        ],
        out_shape=[
            jax.ShapeDtypeStruct((NP, NHID), jnp.float32),
            jax.ShapeDtypeStruct((NP, NHID), jnp.float32),
        ],
    )(c, m, w, b)


def _agg_plain_body(c_ref, m_ref, x_ref):
    cb = c_ref[...]
    deg = jnp.sum(cb, axis=1, keepdims=True)
    invd = 0.5 / jnp.maximum(deg, 1.0)
    y = jnp.dot(cb, m_ref[...], preferred_element_type=jnp.float32, precision=lax.Precision.HIGHEST) * invd
    x_ref[...] = jnp.maximum(y, 0.0)


def _agg_plain(c, m):
    return pl.pallas_call(
        _agg_plain_body,
        grid=(NP // BLK,),
        in_specs=[
            pl.BlockSpec((BLK, NP), lambda i: (i, 0)),
            pl.BlockSpec((NP, NHID), lambda i: (0, 0)),
        ],
        out_specs=pl.BlockSpec((BLK, NHID), lambda i: (i, 0)),
        out_shape=jax.ShapeDtypeStruct((NP, NHID), jnp.float32),
    )(c, m)


def _finalize_body(gw, c_ref, x1_ref, x2_ref, x3_ref, pw1_ref, pb1_ref,
                   pw2_ref, pb2_ref, o_ref):
    i = pl.program_id(0)
    cb = c_ref[...]
    deg = jnp.sum(cb, axis=1, keepdims=True)
    invd = 1.0 / jnp.maximum(deg, 1.0)
    pw1 = pw1_ref[...]
    pw2 = pw2_ref[...]
    pb1 = pb1_ref[0, 0]
    pb2 = pb2_ref[0, 0]
    xs_blk = []
    pcols = []
    gcols = []
    for x_ref in (x1_ref, x2_ref, x3_ref):
        xf = x_ref[...]
        xb = x_ref[pl.ds(i * BLK, BLK), :]
        xs_blk.append(xb)
        pcols.append(jnp.sum(xf * pw2, axis=1, keepdims=True) + pb2)
        gcols.append(jnp.sum(xb * pw1, axis=1, keepdims=True) + pb1)
    p_full = jnp.concatenate(pcols, axis=1)
    sl = jnp.dot(cb, p_full, preferred_element_type=jnp.float32, precision=lax.Precision.HIGHEST) * invd
    g = jnp.concatenate(gcols, axis=1)
    w = ALPHA * g + (1.0 - ALPHA) * sl
    m = jnp.max(w, axis=1, keepdims=True)
    e = jnp.exp(w - m)
    wn = e / jnp.sum(e, axis=1, keepdims=True)
    o_ref[...] = gw * (wn[:, 0:1] * xs_blk[0]
                       + wn[:, 1:2] * xs_blk[1]
                       + wn[:, 2:3] * xs_blk[2])


def _finalize(gw, c, x1, x2, x3, pw1, pb1, pw2, pb2):
    return pl.pallas_call(
        functools.partial(_finalize_body, gw),
        grid=(NP // BLK,),
        in_specs=[
            pl.BlockSpec((BLK, NP), lambda i: (i, 0)),
            pl.BlockSpec((NP, NHID), lambda i: (0, 0)),
            pl.BlockSpec((NP, NHID), lambda i: (0, 0)),
            pl.BlockSpec((NP, NHID), lambda i: (0, 0)),
            pl.BlockSpec((1, NHID), lambda i: (0, 0)),
            pl.BlockSpec((1, 1), lambda i: (0, 0)),
            pl.BlockSpec((1, NHID), lambda i: (0, 0)),
            pl.BlockSpec((1, 1), lambda i: (0, 0)),
        ],
        out_specs=pl.BlockSpec((BLK, NHID), lambda i: (i, 0)),
        out_shape=jax.ShapeDtypeStruct((NP, NHID), jnp.float32),
    )(c, x1, x2, x3, pw1, pb1, pw2, pb2)


def _combine_body(a_ref, b_ref, c_ref, o_ref):
    o_ref[...] = a_ref[...] + b_ref[...] + c_ref[...]


def _combine(a, b, c):
    return pl.pallas_call(
        _combine_body,
        grid=(NP // BLK,),
        in_specs=[pl.BlockSpec((BLK, NHID), lambda i: (i, 0))] * 3,
        out_specs=pl.BlockSpec((BLK, NHID), lambda i: (i, 0)),
        out_shape=jax.ShapeDtypeStruct((NP, NHID), jnp.float32),
    )(a, b, c)


BMLP = 512


def _mlp_body(dr_ref, di_ref, w0_ref, b0_ref, w1_ref, b1_ref, w2_ref, b2_ref,
              w3_ref, b3_ref, emb_ref, o_ref):
    e = dr_ref[...] * di_ref[...]
    emb_ref[...] = e
    h = jnp.maximum(jnp.dot(e, w0_ref[...], preferred_element_type=jnp.float32, precision=lax.Precision.HIGHEST)
                    + b0_ref[...], 0.0)
    h = jnp.maximum(jnp.dot(h, w1_ref[...], preferred_element_type=jnp.float32, precision=lax.Precision.HIGHEST)
                    + b1_ref[...], 0.0)
    h = jnp.maximum(jnp.dot(h, w2_ref[...], preferred_element_type=jnp.float32, precision=lax.Precision.HIGHEST)
                    + b2_ref[...], 0.0)
    o_ref[...] = jnp.dot(h, w3_ref[...], preferred_element_type=jnp.float32, precision=lax.Precision.HIGHEST) + b3_ref[...]


def _mlp(dr, di, w0, b0, w1, b1, w2, b2, w3p, b3p):
    full = lambda r, c: pl.BlockSpec((r, c), lambda i: (0, 0))
    return pl.pallas_call(
        _mlp_body,
        grid=(BATCH // BMLP,),
        in_specs=[
            pl.BlockSpec((BMLP, NHID), lambda i: (i, 0)),
            pl.BlockSpec((BMLP, NHID), lambda i: (i, 0)),
            full(NHID, 1024), full(1, 1024),
            full(1024, 512), full(1, 512),
            full(512, 256), full(1, 256),
            full(256, 128), full(1, 128),
        ],
        out_specs=[
            pl.BlockSpec((BMLP, NHID), lambda i: (i, 0)),
            pl.BlockSpec((BMLP, 128), lambda i: (i, 0)),
        ],
        out_shape=[
            jax.ShapeDtypeStruct((BATCH, NHID), jnp.float32),
            jax.ShapeDtypeStruct((BATCH, 128), jnp.float32),
        ],
    )(dr, di, w0, b0, w1, b1, w2, b2, w3p, b3p)


def kernel(sample, adj, adj_edge_index1, adj_edge_index2, adj_edge_index3,
           W_gc1, b_gc1, W_gc2, b_gc2, W_gc3, b_gc3,
           pool_w1, pool_b1, pool_w2, pool_b2,
           mlp_w0, mlp_b0, mlp_w1, mlp_b1, mlp_w2, mlp_b2, mlp_w3, mlp_b3):
    f32 = jnp.float32

    srcs = jnp.concatenate(
        [jnp.pad(ei[0].astype(jnp.int32), (0, EP - N_EDGES))
         for ei in (adj_edge_index1, adj_edge_index2, adj_edge_index3)])
    dsts = jnp.concatenate(
        [jnp.pad(ei[1].astype(jnp.int32), (0, EP - N_EDGES),
                 constant_values=DST_PAD)
         for ei in (adj_edge_index1, adj_edge_index2, adj_edge_index3)])
    adj_p = jnp.pad(adj.astype(f32), ((0, NP - N_NODES), (0, NP - N_NODES)))
    w1_p = jnp.pad(W_gc1.astype(f32), ((0, 0), (0, NP - N_NODES), (0, 0)))
    b1r = b_gc1.reshape(2, 1, NHID).astype(f32)
    b2r = b_gc2.reshape(2, 1, NHID).astype(f32)
    b3r = b_gc3.reshape(2, 1, NHID).astype(f32)
    pw1 = pool_w1.reshape(1, NHID).astype(f32)
    pw2 = pool_w2.reshape(1, NHID).astype(f32)
    pb1 = pool_b1.reshape(1, 1).astype(f32)
    pb2 = pool_b2.reshape(1, 1).astype(f32)
    s0r = sample[:, 0].astype(jnp.int32).reshape(128, 128)
    s1r = sample[:, 1].astype(jnp.int32).reshape(128, 128)
    w3p = jnp.pad(mlp_w3.astype(f32), ((0, 0), (0, 126)))
    b3p = jnp.pad(mlp_b3.astype(f32), (0, 126)).reshape(1, 128)
    b0r = mlp_b0.reshape(1, 1024).astype(f32)
    b1m = mlp_b1.reshape(1, 512).astype(f32)
    b2m = mlp_b2.reshape(1, 256).astype(f32)

    c_all = _sc_build(srcs, dsts)

    h1 = _mm_h1(adj_p, w1_p, b1r)

    nfs = []
    for g, gw in enumerate((0.3, 0.35, 0.35)):
        cg = c_all[g]
        x1, m2 = _agg_fused(cg, h1, W_gc2.astype(f32), b2r)
        x2, m3 = _agg_fused(cg, m2, W_gc3.astype(f32), b3r)
        x3 = _agg_plain(cg, m3)
        nfs.append(_finalize(gw, cg, x1, x2, x3, pw1, pb1, pw2, pb2))
    nf = _combine(*nfs)

    dr_rows, di_rows = _sc_gather(nf, s0r, s1r)

    emb, outp = _mlp(dr_rows, di_rows, mlp_w0.astype(f32), b0r,
                     mlp_w1.astype(f32), b1m, mlp_w2.astype(f32), b2m, w3p, b3p)
    return emb, outp[:, :2]


# speedup vs baseline: 2.6291x; 1.4391x over previous
import functools
import jax
import jax.numpy as jnp
from jax import lax
from jax.experimental import pallas as pl
from jax.experimental.pallas import tpu as pltpu
from jax.experimental.pallas import tpu_sc as plsc

N_NODES = 2249
N_DRUG = 2033
NHID = 128
ALPHA = 0.9
N_EDGES = 143936
BATCH = 16384

NP = 2304
BLK = 256
E_T = 9088
EP = 16 * E_T
NROWS_I = 71
R = 576
BUF_MAIN = R * NP
BUFLEN = BUF_MAIN + 2048
STRIPE = BUF_MAIN // 16
ZV = STRIPE // 4
DST_PAD = 4000

_SC_MESH = plsc.VectorSubcoreMesh(core_axis_name="c", subcore_axis_name="s")


def _sc_build_body(srcs, dsts, ones_h, zeros_h, c_out,
                   src_v, dst_v, idx1d, ones_v, buf):
    core = lax.axis_index("c")
    sub = lax.axis_index("s")
    pltpu.sync_copy(ones_h, ones_v)
    eoff = sub * E_T
    lane = lax.iota(jnp.int32, 16)
    gbase = BUF_MAIN + sub * 128
    for g in range(3):
        pltpu.sync_copy(srcs.at[pl.ds(g * EP + eoff, E_T)], src_v)
        pltpu.sync_copy(dsts.at[pl.ds(g * EP + eoff, E_T)], dst_v)
        for ch_i in range(2):
            chunk = core * 2 + ch_i
            base = chunk * R
            pltpu.sync_copy(zeros_h, buf.at[pl.ds(sub * STRIPE, STRIPE)])
            plsc.subcore_barrier()

            def idx_body(b, carry):
                for k in range(8):
                    e = b * 128 + k * 16
                    s = src_v[pl.ds(e, 16)]
                    d = dst_v[pl.ds(e, 16)]
                    rel = d - base
                    ok = (rel >= 0) & (rel < R)
                    flat = rel * NP + s
                    flat = jnp.where(ok, flat, gbase + k * 16 + lane)
                    idx1d[pl.ds(e, 16)] = flat
                return carry
            lax.fori_loop(0, NROWS_I, idx_body, 0)

            pltpu.sync_copy(ones_v, buf.at[idx1d], add=True)
            plsc.subcore_barrier()

            dst_off = g * NP * NP + base * NP + sub * STRIPE
            pltpu.sync_copy(buf.at[pl.ds(sub * STRIPE, STRIPE)],
                            c_out.at[pl.ds(dst_off, STRIPE)])
            plsc.subcore_barrier()


def _sc_build(srcs, dsts):
    ones_h = jnp.ones((E_T,), jnp.float32)
    zeros_h = jnp.zeros((STRIPE,), jnp.float32)
    k = pl.kernel(
        _sc_build_body,
        out_type=jax.ShapeDtypeStruct((3 * NP * NP,), jnp.float32),
        mesh=_SC_MESH,
        scratch_types=[
            pltpu.VMEM((E_T,), jnp.int32),
            pltpu.VMEM((E_T,), jnp.int32),
            pltpu.VMEM((E_T,), jnp.int32),
            pltpu.VMEM((E_T,), jnp.float32),
            pltpu.VMEM_SHARED((BUFLEN,), jnp.float32),
        ],
    )
    return k(srcs, dsts, ones_h, zeros_h).reshape(3, NP, NP)


def _sc_gather_body(nf, s0r, s1r, dr, di, idx_s, rows_v, sem):
    core = lax.axis_index("c")
    sub = lax.axis_index("s")
    wid = sub * 2 + core
    pltpu.sync_copy(s0r.at[pl.ds(wid * 4, 4)], idx_s.at[pl.ds(0, 4)])
    pltpu.sync_copy(s1r.at[pl.ds(wid * 4, 4)], idx_s.at[pl.ds(4, 4)])

    def addb(i, carry):
        r = 4 + i // 8
        c = (i % 8) * 16
        idx_s[r, pl.ds(c, 16)] = idx_s[r, pl.ds(c, 16)] + N_DRUG
        return carry
    lax.fori_loop(0, 32, addb, 0)

    for t, out in ((0, dr), (1, di)):
        for b in range(4):
            pltpu.async_copy(nf.at[idx_s.at[t * 4 + b]], rows_v, sem).wait()
            pltpu.sync_copy(rows_v, out.at[pl.ds(wid * 512 + b * 128, 128)])


def _sc_gather(nf, s0r, s1r):
    k = pl.kernel(
        _sc_gather_body,
        out_type=(jax.ShapeDtypeStruct((BATCH, NHID), jnp.float32),
                  jax.ShapeDtypeStruct((BATCH, NHID), jnp.float32)),
        mesh=_SC_MESH,
        scratch_types=[
            pltpu.VMEM((8, 128), jnp.int32),
            pltpu.VMEM((128, NHID), jnp.float32),
            pltpu.SemaphoreType.DMA,
        ],
    )
    return k(nf, s0r, s1r)


def _dot3(a, b):
    ah = a.astype(jnp.bfloat16)
    al = (a - ah.astype(jnp.float32)).astype(jnp.bfloat16)
    bh = b.astype(jnp.bfloat16)
    bl = (b - bh.astype(jnp.float32)).astype(jnp.bfloat16)
    mm = functools.partial(jnp.dot, preferred_element_type=jnp.float32)
    return (mm(ah, bl) + mm(al, bh)) + mm(ah, bh)


def _h1_body(a_ref, w_ref, b_ref, o_ref):
    w = w_ref[0] + w_ref[1]
    b = b_ref[0] + b_ref[1]
    o_ref[...] = _dot3(a_ref[...], w) + b


def _mm_h1(adj_p, w_p, b):
    return pl.pallas_call(
        _h1_body,
        grid=(NP // BLK,),
        in_specs=[
            pl.BlockSpec((BLK, NP), lambda i: (i, 0)),
            pl.BlockSpec((2, NP, NHID), lambda i: (0, 0, 0)),
            pl.BlockSpec((2, 1, NHID), lambda i: (0, 0, 0)),
        ],
        out_specs=pl.BlockSpec((BLK, NHID), lambda i: (i, 0)),
        out_shape=jax.ShapeDtypeStruct((NP, NHID), jnp.float32),
    )(adj_p, w_p, b)


def _agg_core(c_ref, m):
    cb = c_ref[0]
    deg = jnp.sum(cb, axis=1, keepdims=True)
    invd = 0.5 / jnp.maximum(deg, 1.0)
    return jnp.maximum(_dot3(cb, m) * invd, 0.0)


def _agg_round_body(c_ref, m_ref, w_ref, b_ref, x_ref, mn_ref):
    x = _agg_core(c_ref, m_ref[0])
    x_ref[0] = x
    w = w_ref[0] + w_ref[1]
    mn_ref[0] = _dot3(x, w) + (b_ref[0] + b_ref[1])


def _agg_round(c_all, m, w, b, m_shared):
    if m_shared:
        m_spec = pl.BlockSpec((1, NP, NHID), lambda g, i: (0, 0, 0))
    else:
        m_spec = pl.BlockSpec((1, NP, NHID), lambda g, i: (g, 0, 0))
    return pl.pallas_call(
        _agg_round_body,
        grid=(3, NP // BLK),
        in_specs=[
            pl.BlockSpec((1, BLK, NP), lambda g, i: (g, i, 0)),
            m_spec,
            pl.BlockSpec((2, NHID, NHID), lambda g, i: (0, 0, 0)),
            pl.BlockSpec((2, 1, NHID), lambda g, i: (0, 0, 0)),
        ],
        out_specs=[
            pl.BlockSpec((1, BLK, NHID), lambda g, i: (g, i, 0)),
            pl.BlockSpec((1, BLK, NHID), lambda g, i: (g, i, 0)),
        ],
        out_shape=[
            jax.ShapeDtypeStruct((3, NP, NHID), jnp.float32),
            jax.ShapeDtypeStruct((3, NP, NHID), jnp.float32),
        ],
    )(c_all, m, w, b)


def _agg_last_body(c_ref, m_ref, x_ref):
    x_ref[0] = _agg_core(c_ref, m_ref[0])


def _agg_last(c_all, m):
    return pl.pallas_call(
        _agg_last_body,
        grid=(3, NP // BLK),
        in_specs=[
            pl.BlockSpec((1, BLK, NP), lambda g, i: (g, i, 0)),
            pl.BlockSpec((1, NP, NHID), lambda g, i: (g, 0, 0)),
        ],
        out_specs=pl.BlockSpec((1, BLK, NHID), lambda g, i: (g, i, 0)),
        out_shape=jax.ShapeDtypeStruct((3, NP, NHID), jnp.float32),
    )(c_all, m)


def _finalize_body(c_ref, x1_ref, x2_ref, x3_ref, pw1_ref, pb1_ref,
                   pw2_ref, pb2_ref, o_ref):
    g = pl.program_id(0)
    i = pl.program_id(1)
    gw = jnp.where(g == 0, 0.3, 0.35)
    cb = c_ref[0]
    deg = jnp.sum(cb, axis=1, keepdims=True)
    invd = 1.0 / jnp.maximum(deg, 1.0)
    pw1 = pw1_ref[...]
    pw2 = pw2_ref[...]
    pb1 = pb1_ref[0, 0]
    pb2 = pb2_ref[0, 0]
    xs_blk = []
    pcols = []
    gcols = []
    for x_ref in (x1_ref, x2_ref, x3_ref):
        xf = x_ref[0]
        xb = x_ref[0, pl.ds(i * BLK, BLK), :]
        xs_blk.append(xb)
        pcols.append(jnp.sum(xf * pw2, axis=1, keepdims=True) + pb2)
        gcols.append(jnp.sum(xb * pw1, axis=1, keepdims=True) + pb1)
    p_full = jnp.concatenate(pcols, axis=1)
    sl = _dot3(cb, p_full) * invd
    gsc = jnp.concatenate(gcols, axis=1)
    w = ALPHA * gsc + (1.0 - ALPHA) * sl
    m = jnp.max(w, axis=1, keepdims=True)
    e = jnp.exp(w - m)
    wn = e / jnp.sum(e, axis=1, keepdims=True)
    o_ref[0] = gw * (wn[:, 0:1] * xs_blk[0]
                     + wn[:, 1:2] * xs_blk[1]
                     + wn[:, 2:3] * xs_blk[2])


def _finalize_all(c_all, x1, x2, x3, pw1, pb1, pw2, pb2):
    xspec = pl.BlockSpec((1, NP, NHID), lambda g, i: (g, 0, 0))
    return pl.pallas_call(
        _finalize_body,
        grid=(3, NP // BLK),
        in_specs=[
            pl.BlockSpec((1, BLK, NP), lambda g, i: (g, i, 0)),
            xspec, xspec, xspec,
            pl.BlockSpec((1, NHID), lambda g, i: (0, 0)),
            pl.BlockSpec((1, 1), lambda g, i: (0, 0)),
            pl.BlockSpec((1, NHID), lambda g, i: (0, 0)),
            pl.BlockSpec((1, 1), lambda g, i: (0, 0)),
        ],
        out_specs=pl.BlockSpec((1, BLK, NHID), lambda g, i: (g, i, 0)),
        out_shape=jax.ShapeDtypeStruct((3, NP, NHID), jnp.float32),
    )(c_all, x1, x2, x3, pw1, pb1, pw2, pb2)


def _combine_body(a_ref, o_ref):
    o_ref[...] = a_ref[0] + a_ref[1] + a_ref[2]


def _combine(nf_all):
    return pl.pallas_call(
        _combine_body,
        grid=(NP // BLK,),
        in_specs=[pl.BlockSpec((3, BLK, NHID), lambda i: (0, i, 0))],
        out_specs=pl.BlockSpec((BLK, NHID), lambda i: (i, 0)),
        out_shape=jax.ShapeDtypeStruct((NP, NHID), jnp.float32),
    )(nf_all)


BMLP = 512


def _mlp_body(dr_ref, di_ref, w0_ref, b0_ref, w1_ref, b1_ref, w2_ref, b2_ref,
              w3_ref, b3_ref, emb_ref, o_ref):
    e = dr_ref[...] * di_ref[...]
    emb_ref[...] = e
    h = jnp.maximum(_dot3(e, w0_ref[...]) + b0_ref[...], 0.0)
    h = jnp.maximum(_dot3(h, w1_ref[...]) + b1_ref[...], 0.0)
    h = jnp.maximum(_dot3(h, w2_ref[...]) + b2_ref[...], 0.0)
    o_ref[...] = _dot3(h, w3_ref[...]) + b3_ref[...]


def _mlp(dr, di, w0, b0, w1, b1, w2, b2, w3p, b3p):
    full = lambda r, c: pl.BlockSpec((r, c), lambda i: (0, 0))
    return pl.pallas_call(
        _mlp_body,
        grid=(BATCH // BMLP,),
        in_specs=[
            pl.BlockSpec((BMLP, NHID), lambda i: (i, 0)),
            pl.BlockSpec((BMLP, NHID), lambda i: (i, 0)),
            full(NHID, 1024), full(1, 1024),
            full(1024, 512), full(1, 512),
            full(512, 256), full(1, 256),
            full(256, 128), full(1, 128),
        ],
        out_specs=[
            pl.BlockSpec((BMLP, NHID), lambda i: (i, 0)),
            pl.BlockSpec((BMLP, 128), lambda i: (i, 0)),
        ],
        out_shape=[
            jax.ShapeDtypeStruct((BATCH, NHID), jnp.float32),
            jax.ShapeDtypeStruct((BATCH, 128), jnp.float32),
        ],
    )(dr, di, w0, b0, w1, b1, w2, b2, w3p, b3p)


def kernel(sample, adj, adj_edge_index1, adj_edge_index2, adj_edge_index3,
           W_gc1, b_gc1, W_gc2, b_gc2, W_gc3, b_gc3,
           pool_w1, pool_b1, pool_w2, pool_b2,
           mlp_w0, mlp_b0, mlp_w1, mlp_b1, mlp_w2, mlp_b2, mlp_w3, mlp_b3):
    f32 = jnp.float32

    srcs = jnp.concatenate(
        [jnp.pad(ei[0].astype(jnp.int32), (0, EP - N_EDGES))
         for ei in (adj_edge_index1, adj_edge_index2, adj_edge_index3)])
    dsts = jnp.concatenate(
        [jnp.pad(ei[1].astype(jnp.int32), (0, EP - N_EDGES),
                 constant_values=DST_PAD)
         for ei in (adj_edge_index1, adj_edge_index2, adj_edge_index3)])
    adj_p = jnp.pad(adj.astype(f32), ((0, NP - N_NODES), (0, NP - N_NODES)))
    w1_p = jnp.pad(W_gc1.astype(f32), ((0, 0), (0, NP - N_NODES), (0, 0)))
    b1r = b_gc1.reshape(2, 1, NHID).astype(f32)
    b2r = b_gc2.reshape(2, 1, NHID).astype(f32)
    b3r = b_gc3.reshape(2, 1, NHID).astype(f32)
    pw1 = pool_w1.reshape(1, NHID).astype(f32)
    pw2 = pool_w2.reshape(1, NHID).astype(f32)
    pb1 = pool_b1.reshape(1, 1).astype(f32)
    pb2 = pool_b2.reshape(1, 1).astype(f32)
    s0r = sample[:, 0].astype(jnp.int32).reshape(128, 128)
    s1r = sample[:, 1].astype(jnp.int32).reshape(128, 128)
    w3p = jnp.pad(mlp_w3.astype(f32), ((0, 0), (0, 126)))
    b3p = jnp.pad(mlp_b3.astype(f32), (0, 126)).reshape(1, 128)
    b0r = mlp_b0.reshape(1, 1024).astype(f32)
    b1m = mlp_b1.reshape(1, 512).astype(f32)
    b2m = mlp_b2.reshape(1, 256).astype(f32)

    c_all = _sc_build(srcs, dsts)

    h1 = _mm_h1(adj_p, w1_p, b1r)

    h1b = h1.reshape(1, NP, NHID)
    x1, m2 = _agg_round(c_all, h1b, W_gc2.astype(f32), b2r, m_shared=True)
    x2, m3 = _agg_round(c_all, m2, W_gc3.astype(f32), b3r, m_shared=False)
    x3 = _agg_last(c_all, m3)
    nf_all = _finalize_all(c_all, x1, x2, x3, pw1, pb1, pw2, pb2)
    nf = _combine(nf_all)

    dr_rows, di_rows = _sc_gather(nf, s0r, s1r)

    emb, outp = _mlp(dr_rows, di_rows, mlp_w0.astype(f32), b0r,
                     mlp_w1.astype(f32), b1m, mlp_w2.astype(f32), b2m, w3p, b3p)
    return emb, outp[:, :2]
